# baseline (speedup 1.0000x reference)
import functools

import jax
import jax.numpy as jnp
from jax import lax
from jax.experimental import pallas as pl
from jax.experimental.pallas import tpu as pltpu
from jax.experimental.pallas import tpu_sc as plsc

B, C, H, W = 16, 512, 32, 32
HW = H * W
EMBED_DIM, TEXT_DIM, NUM_TOPK = 256, 768, 100


def _density_body(feat_ref, wd_ref, bd_ref, dens_ref):
    f = feat_ref[0]
    wd = wd_ref[...]
    lg = jnp.dot(wd, f, preferred_element_type=jnp.float32) + bd_ref[...]
    dens_ref[0] = jax.nn.sigmoid(lg)


_NCHUNK = HW // 16


def _sc_mask_body(dens_hbm, mask_hbm, row_v, keys_v, mask_v,
                  acc_v, mid_v, lo_v, hi_v, need_v, red_v):
    wid = lax.axis_index("s") * 2 + lax.axis_index("c")
    zero16 = jnp.zeros((16,), jnp.int32)
    one16 = jnp.ones((16,), jnp.int32)

    @pl.when(wid < B)
    def _():
        pltpu.sync_copy(dens_hbm.at[wid], row_v)

        def bc(j, _):
            keys_v[pl.ds(j * 16, 16)] = lax.bitcast_convert_type(
                row_v[pl.ds(j * 16, 16)], jnp.int32)
            return 0
        lax.fori_loop(0, _NCHUNK, bc, 0)

        lo_v[...] = jnp.full((16,), -1, jnp.int32)
        hi_v[...] = jnp.full((16,), 2 ** 30, jnp.int32)

        def vstep(i, _):
            lo = lo_v[...]
            hi = hi_v[...]
            mid_v[...] = lo + lax.shift_right_arithmetic(hi - lo, 1)
            acc_v[...] = zero16

            def chunk(j, __):
                t = mid_v[...]
                acc_v[...] = acc_v[...] + jnp.where(
                    keys_v[pl.ds(j * 16, 16)] > t, one16, zero16)
                return 0
            lax.fori_loop(0, _NCHUNK, chunk, 0)

            red_v[pl.ds(0, 16)] = acc_v[...]
            red_v[pl.ds(16, 16)] = acc_v[...]
            for s in (8, 4, 2, 1):
                cur = red_v[pl.ds(0, 16)]
                rot = red_v[pl.ds(s, 16)]
                nxt = cur + rot
                red_v[pl.ds(0, 16)] = nxt
                red_v[pl.ds(16, 16)] = nxt
            cnt = red_v[pl.ds(0, 16)]
            k_count = jnp.full((16,), NUM_TOPK, jnp.int32)
            pred = cnt < k_count
            mid = mid_v[...]
            lo_v[...] = jnp.where(pred, lo, mid)
            hi_v[...] = jnp.where(pred, mid, hi)
            return 0
        lax.fori_loop(0, 31, vstep, 0)
        mid_v[...] = hi_v[...]

        acc_v[...] = zero16

        def ngt(j, __):
            t = mid_v[...]
            acc_v[...] = acc_v[...] + jnp.where(
                keys_v[pl.ds(j * 16, 16)] > t, one16, zero16)
            return 0
        lax.fori_loop(0, _NCHUNK, ngt, 0)
        red_v[pl.ds(0, 16)] = acc_v[...]
        red_v[pl.ds(16, 16)] = acc_v[...]
        for s in (8, 4, 2, 1):
            nxt = red_v[pl.ds(0, 16)] + red_v[pl.ds(s, 16)]
            red_v[pl.ds(0, 16)] = nxt
            red_v[pl.ds(16, 16)] = nxt
        need_v[...] = jnp.full((16,), NUM_TOPK, jnp.int32) - red_v[pl.ds(0, 16)]

        acc_v[...] = zero16

        def w(j, _):
            k = keys_v[pl.ds(j * 16, 16)]
            t = mid_v[...]
            need = need_v[...]
            eq = k == t
            e = jnp.where(eq, one16, zero16)
            p = e
            for s in (1, 2, 4, 8):
                red_v[pl.ds(0, 16)] = zero16
                red_v[pl.ds(s, 16)] = p
                p = p + red_v[pl.ds(0, 16)]
            pin = p + acc_v[...]
            m = jnp.logical_or(k > t, jnp.logical_and(eq, pin <= need))
            mask_v[pl.ds(j * 16, 16)] = jnp.where(m, 1.0, 0.0)
            red_v[pl.ds(0, 16)] = pin
            red_v[pl.ds(16, 16)] = pin
            for s in (8, 4, 2, 1):
                nxt = jnp.maximum(red_v[pl.ds(0, 16)], red_v[pl.ds(s, 16)])
                red_v[pl.ds(0, 16)] = nxt
                red_v[pl.ds(16, 16)] = nxt
            acc_v[...] = red_v[pl.ds(0, 16)]
            return 0
        lax.fori_loop(0, _NCHUNK, w, 0)
        pltpu.sync_copy(mask_v, mask_hbm.at[wid])


def _sc_mask(density):
    mesh = plsc.VectorSubcoreMesh(core_axis_name="c", subcore_axis_name="s")
    return pl.kernel(
        _sc_mask_body,
        out_type=jax.ShapeDtypeStruct((B, HW), jnp.float32),
        mesh=mesh,
        scratch_types=[
            pltpu.VMEM((HW,), jnp.float32),
            pltpu.VMEM((HW,), jnp.int32),
            pltpu.VMEM((HW,), jnp.float32),
            pltpu.VMEM((16,), jnp.int32),
            pltpu.VMEM((16,), jnp.int32),
            pltpu.VMEM((16,), jnp.int32),
            pltpu.VMEM((16,), jnp.int32),
            pltpu.VMEM((16,), jnp.int32),
            pltpu.VMEM((32,), jnp.int32),
        ],
    )(density)


def _fusion_body(feat_ref, mask_ref, wsp_ref, wout_ref, wtext_ref, temb_ref,
                 bsp_ref, btext_ref, bout_ref, out_ref):
    f = feat_ref[0]
    m = mask_ref[0]
    tcol = (jnp.dot(wtext_ref[...], temb_ref[...],
                    preferred_element_type=jnp.float32)
            + btext_ref[...] + bsp_ref[...])
    z1 = jnp.dot(wsp_ref[...].astype(jnp.bfloat16), f.astype(jnp.bfloat16),
                 preferred_element_type=jnp.float32) + tcol
    z2 = jnp.dot(wout_ref[...].astype(jnp.bfloat16), z1.astype(jnp.bfloat16),
                 preferred_element_type=jnp.float32) + bout_ref[...]
    out_ref[0] = jnp.where(m > 0.0, z2, f)


@functools.partial(jax.jit, static_argnames=())
def kernel(feat, text_emb, Wd, bd, W_sp, b_sp, W_text, b_text, W_out, b_out):
    b, c, h, w = feat.shape
    feat3 = feat.reshape(b, c, h * w)

    density = pl.pallas_call(
        _density_body,
        grid=(b,),
        in_specs=[
            pl.BlockSpec((1, c, h * w), lambda i: (i, 0, 0)),
            pl.BlockSpec((1, c), lambda i: (0, 0)),
            pl.BlockSpec((1, 1), lambda i: (0, 0)),
        ],
        out_specs=pl.BlockSpec((1, 1, h * w), lambda i: (i, 0, 0)),
        out_shape=jax.ShapeDtypeStruct((b, 1, h * w), jnp.float32),
        compiler_params=pltpu.CompilerParams(
            dimension_semantics=("arbitrary",)),
    )(feat3, Wd.reshape(1, c), bd.reshape(1, 1))

    mask = _sc_mask(density.reshape(b, h * w))

    out = pl.pallas_call(
        _fusion_body,
        grid=(b,),
        in_specs=[
            pl.BlockSpec((1, c, h * w), lambda i: (i, 0, 0)),
            pl.BlockSpec((1, 1, h * w), lambda i: (i, 0, 0)),
            pl.BlockSpec((EMBED_DIM, c), lambda i: (0, 0)),
            pl.BlockSpec((c, EMBED_DIM), lambda i: (0, 0)),
            pl.BlockSpec((EMBED_DIM, TEXT_DIM), lambda i: (0, 0)),
            pl.BlockSpec((TEXT_DIM, 1), lambda i: (0, 0)),
            pl.BlockSpec((EMBED_DIM, 1), lambda i: (0, 0)),
            pl.BlockSpec((EMBED_DIM, 1), lambda i: (0, 0)),
            pl.BlockSpec((c, 1), lambda i: (0, 0)),
        ],
        out_specs=pl.BlockSpec((1, c, h * w), lambda i: (i, 0, 0)),
        out_shape=jax.ShapeDtypeStruct((b, c, h * w), jnp.float32),
        compiler_params=pltpu.CompilerParams(
            dimension_semantics=("arbitrary",)),
    )(feat3, mask.reshape(b, 1, h * w), W_sp, W_out, W_text,
      text_emb.reshape(TEXT_DIM, 1), b_sp.reshape(EMBED_DIM, 1),
      b_text.reshape(EMBED_DIM, 1), b_out.reshape(c, 1))

    return out.reshape(b, c, h, w)
